# baseline (speedup 1.0000x reference)
import functools

import jax
import jax.numpy as jnp
from jax import lax
from jax.experimental import pallas as pl
from jax.experimental.pallas import tpu as pltpu
from jax.experimental.pallas import tpu_sc as plsc

NC = 2
NS = 16
NW = NC * NS
L = 16

_VOCAB = 100000
_RESERVOIR = 8
_FEATURES = 64
_N_TOKENS = 4096 * 50
_NDREG = _FEATURES // L

_CHUNK_ROWS = 128
_CHUNK_IDX = _CHUNK_ROWS * _RESERVOIR
_P1_CHUNKS_PER_W = 25
_N_CHUNKS_PAD = _P1_CHUNKS_PER_W * NW
_VOCAB_PAD = _N_CHUNKS_PAD * _CHUNK_ROWS
_P1_NBUF = 5
_P1_LOOK = 2
_P1_OUTER = _P1_CHUNKS_PER_W // _P1_NBUF

_TOK_CHUNK = 128
_TOK_PER_W = _N_TOKENS // NW
_P2_CHUNKS_PER_W = _TOK_PER_W // _TOK_CHUNK
_P2_NBUF = 10
_P2_LOOK = 4
_P2_OUTER = _P2_CHUNKS_PER_W // _P2_NBUF


def _build_reduced_table(t3_hbm, e_hbm, r_hbm, idx_v, acc_v, gsem, wsem,
                         isem):
    wid = lax.axis_index("s") * NC + lax.axis_index("c")
    c0 = wid * _P1_CHUNKS_PER_W

    def idx_cp(chunk, b):
        return pltpu.make_async_copy(t3_hbm.at[chunk], idx_v.at[b], isem[b])

    def fire_gathers(b):
        zero = jnp.zeros((L,), jnp.float32)
        for j in range(_CHUNK_ROWS):
            for d in range(_NDREG):
                acc_v[b, j, pl.ds(d * L, L)] = zero
        for r in range(_RESERVOIR):
            pltpu.async_copy(e_hbm.at[idx_v.at[b, r]], acc_v.at[b], gsem[b],
                             add=True)

    def wait_gathers(b):
        for _r in range(_RESERVOIR):
            pltpu.make_async_copy(e_hbm.at[idx_v.at[b, 0]], acc_v.at[b],
                                  gsem[b]).wait()

    def wback(chunk, b):
        return pltpu.make_async_copy(
            acc_v.at[b], r_hbm.at[pl.ds(chunk * _CHUNK_ROWS, _CHUNK_ROWS)],
            wsem[b])

    for b in range(_P1_NBUF):
        idx_cp(c0 + b, b).start()
    for b in range(_P1_LOOK):
        idx_cp(c0 + b, b).wait()
        fire_gathers(b)

    def body(o, _):
        for b in range(_P1_NBUF):
            i = o * _P1_NBUF + b
            c = c0 + i
            wait_gathers(b)
            @pl.when(i + _P1_NBUF < _P1_CHUNKS_PER_W)
            def _():
                idx_cp(c + _P1_NBUF, b).start()
            bj = (b + _P1_LOOK) % _P1_NBUF
            @pl.when(i + _P1_LOOK < _P1_CHUNKS_PER_W)
            def _():
                idx_cp(c + _P1_LOOK, bj).wait()
                @pl.when(i + _P1_LOOK >= _P1_NBUF)
                def _():
                    wback(c + _P1_LOOK - _P1_NBUF, bj).wait()
                fire_gathers(bj)
            wback(c, b).start()
        return 0

    lax.fori_loop(0, _P1_OUTER, body, 0)
    for b in range(_P1_NBUF):
        wback(c0 + _P1_CHUNKS_PER_W - _P1_NBUF + b, b).wait()


def _apply_table(base_hbm, r_hbm, out_hbm, idx_v, rows_v, gsem, wsem, isem):
    wid = lax.axis_index("s") * NC + lax.axis_index("c")
    t0w = wid * _TOK_PER_W

    def idx_cp(i, b):
        return pltpu.make_async_copy(
            base_hbm.at[pl.ds(t0w + i * _TOK_CHUNK, _TOK_CHUNK)],
            idx_v.at[b], isem[b])

    def gather(b):
        return pltpu.make_async_copy(r_hbm.at[idx_v.at[b]], rows_v.at[b],
                                     gsem[b])

    def wback(i, b):
        return pltpu.make_async_copy(
            rows_v.at[b],
            out_hbm.at[pl.ds(t0w + i * _TOK_CHUNK, _TOK_CHUNK)], wsem[b])

    for b in range(_P2_NBUF):
        idx_cp(b, b).start()
    for b in range(_P2_LOOK):
        idx_cp(b, b).wait()
        gather(b).start()

    def body(o, _):
        for b in range(_P2_NBUF):
            i = o * _P2_NBUF + b
            gather(b).wait()
            @pl.when(i + _P2_NBUF < _P2_CHUNKS_PER_W)
            def _():
                idx_cp(i + _P2_NBUF, b).start()
            bj = (b + _P2_LOOK) % _P2_NBUF
            @pl.when(i + _P2_LOOK < _P2_CHUNKS_PER_W)
            def _():
                idx_cp(i + _P2_LOOK, bj).wait()
                @pl.when(i + _P2_LOOK >= _P2_NBUF)
                def _():
                    wback(i + _P2_LOOK - _P2_NBUF, bj).wait()
                gather(bj).start()
            wback(i, b).start()
        return 0

    lax.fori_loop(0, _P2_OUTER, body, 0)
    for b in range(_P2_NBUF):
        wback(_P2_CHUNKS_PER_W - _P2_NBUF + b, b).wait()


@jax.jit
def kernel(base_indices, token_reservoir_lookup, embedding_table):
    mesh = plsc.VectorSubcoreMesh(core_axis_name="c", subcore_axis_name="s")
    cparams = pltpu.CompilerParams(use_tc_tiling_on_sc=False)

    t_flat = token_reservoir_lookup.reshape(-1)
    t_pad = jnp.concatenate(
        [t_flat,
         jnp.zeros((_VOCAB_PAD - _VOCAB) * _RESERVOIR, jnp.int32)]
    ).reshape(_N_CHUNKS_PAD, _CHUNK_ROWS, _RESERVOIR).transpose(0, 2, 1)

    build = functools.partial(
        pl.kernel,
        mesh=mesh,
        compiler_params=cparams,
        out_type=jax.ShapeDtypeStruct((_VOCAB_PAD, _FEATURES), jnp.float32),
        scratch_types=[
            pltpu.VMEM((_P1_NBUF, _RESERVOIR, _CHUNK_ROWS), jnp.int32),
            pltpu.VMEM((_P1_NBUF, _CHUNK_ROWS, _FEATURES), jnp.float32),
            [pltpu.SemaphoreType.DMA] * _P1_NBUF,
            [pltpu.SemaphoreType.DMA] * _P1_NBUF,
            [pltpu.SemaphoreType.DMA] * _P1_NBUF,
        ],
    )(_build_reduced_table)
    reduced = build(t_pad, embedding_table)

    apply_k = functools.partial(
        pl.kernel,
        mesh=mesh,
        compiler_params=cparams,
        out_type=jax.ShapeDtypeStruct((_N_TOKENS, _FEATURES), jnp.float32),
        scratch_types=[
            pltpu.VMEM((_P2_NBUF, _TOK_CHUNK), jnp.int32),
            pltpu.VMEM((_P2_NBUF, _TOK_CHUNK, _FEATURES), jnp.float32),
            [pltpu.SemaphoreType.DMA] * _P2_NBUF,
            [pltpu.SemaphoreType.DMA] * _P2_NBUF,
            [pltpu.SemaphoreType.DMA] * _P2_NBUF,
        ],
    )(_apply_table)
    out = apply_k(base_indices.reshape(-1), reduced)
    return out.reshape(*base_indices.shape, _FEATURES)


# speedup vs baseline: 1.7997x; 1.7997x over previous
import functools

import jax
import jax.numpy as jnp
from jax import lax
from jax.experimental import pallas as pl
from jax.experimental.pallas import tpu as pltpu
from jax.experimental.pallas import tpu_sc as plsc

NC = 2
NS = 16
NW = NC * NS
L = 16

_VOCAB = 100000
_RESERVOIR = 8
_FEATURES = 64
_N_TOKENS = 4096 * 50
_NDREG = _FEATURES // L

_CHUNK_ROWS = 32
_CHUNK_IDX = _CHUNK_ROWS * _RESERVOIR
_P1_CHUNKS_PER_W = 98
_N_CHUNKS_PAD = _P1_CHUNKS_PER_W * NW
_VOCAB_PAD = _N_CHUNKS_PAD * _CHUNK_ROWS
_P1_NBUF = 7
_P1_LOOK = 3
_P1_OUTER = _P1_CHUNKS_PER_W // _P1_NBUF

_TOK_CHUNK = 128
_TOK_PER_W = _N_TOKENS // NW
_P2_CHUNKS_PER_W = _TOK_PER_W // _TOK_CHUNK
_P2_NBUF = 10
_P2_LOOK = 4
_P2_OUTER = _P2_CHUNKS_PER_W // _P2_NBUF


def _build_reduced_table(t3_hbm, e_hbm, r_hbm, idx_v, acc_v, gsem, wsem,
                         isem):
    wid = lax.axis_index("s") * NC + lax.axis_index("c")
    c0 = wid * _P1_CHUNKS_PER_W

    def idx_cp(chunk, b):
        return pltpu.make_async_copy(t3_hbm.at[chunk], idx_v.at[b], isem[b])

    def fire_gathers(b):
        zero = jnp.zeros((L,), jnp.float32)
        for j in range(_CHUNK_ROWS):
            for d in range(_NDREG):
                acc_v[b, j, pl.ds(d * L, L)] = zero
        for r in range(_RESERVOIR):
            pltpu.async_copy(e_hbm.at[idx_v.at[b, r]], acc_v.at[b], gsem[b],
                             add=True)

    def wait_gathers(b):
        for _r in range(_RESERVOIR):
            pltpu.make_async_copy(e_hbm.at[idx_v.at[b, 0]], acc_v.at[b],
                                  gsem[b]).wait()

    def wback(chunk, b):
        return pltpu.make_async_copy(
            acc_v.at[b], r_hbm.at[pl.ds(chunk * _CHUNK_ROWS, _CHUNK_ROWS)],
            wsem[b])

    for b in range(_P1_NBUF):
        idx_cp(c0 + b, b).start()
    for b in range(_P1_LOOK):
        idx_cp(c0 + b, b).wait()
        fire_gathers(b)

    def body(o, _):
        for b in range(_P1_NBUF):
            i = o * _P1_NBUF + b
            c = c0 + i
            wait_gathers(b)
            @pl.when(i + _P1_NBUF < _P1_CHUNKS_PER_W)
            def _():
                idx_cp(c + _P1_NBUF, b).start()
            bj = (b + _P1_LOOK) % _P1_NBUF
            @pl.when(i + _P1_LOOK < _P1_CHUNKS_PER_W)
            def _():
                idx_cp(c + _P1_LOOK, bj).wait()
                @pl.when(i + _P1_LOOK >= _P1_NBUF)
                def _():
                    wback(c + _P1_LOOK - _P1_NBUF, bj).wait()
                fire_gathers(bj)
            wback(c, b).start()
        return 0

    lax.fori_loop(0, _P1_OUTER, body, 0)
    for b in range(_P1_NBUF):
        wback(c0 + _P1_CHUNKS_PER_W - _P1_NBUF + b, b).wait()


def _apply_table(base_hbm, r_hbm, out_hbm, idx_v, rows_v, gsem, wsem, isem):
    wid = lax.axis_index("s") * NC + lax.axis_index("c")
    t0w = wid * _TOK_PER_W

    def idx_cp(i, b):
        return pltpu.make_async_copy(
            base_hbm.at[pl.ds(t0w + i * _TOK_CHUNK, _TOK_CHUNK)],
            idx_v.at[b], isem[b])

    def gather(b):
        return pltpu.make_async_copy(r_hbm.at[idx_v.at[b]], rows_v.at[b],
                                     gsem[b])

    def wback(i, b):
        return pltpu.make_async_copy(
            rows_v.at[b],
            out_hbm.at[pl.ds(t0w + i * _TOK_CHUNK, _TOK_CHUNK)], wsem[b])

    for b in range(_P2_NBUF):
        idx_cp(b, b).start()
    for b in range(_P2_LOOK):
        idx_cp(b, b).wait()
        gather(b).start()

    def body(o, _):
        for b in range(_P2_NBUF):
            i = o * _P2_NBUF + b
            gather(b).wait()
            @pl.when(i + _P2_NBUF < _P2_CHUNKS_PER_W)
            def _():
                idx_cp(i + _P2_NBUF, b).start()
            bj = (b + _P2_LOOK) % _P2_NBUF
            @pl.when(i + _P2_LOOK < _P2_CHUNKS_PER_W)
            def _():
                idx_cp(i + _P2_LOOK, bj).wait()
                @pl.when(i + _P2_LOOK >= _P2_NBUF)
                def _():
                    wback(i + _P2_LOOK - _P2_NBUF, bj).wait()
                gather(bj).start()
            wback(i, b).start()
        return 0

    lax.fori_loop(0, _P2_OUTER, body, 0)
    for b in range(_P2_NBUF):
        wback(_P2_CHUNKS_PER_W - _P2_NBUF + b, b).wait()


@jax.jit
def kernel(base_indices, token_reservoir_lookup, embedding_table):
    mesh = plsc.VectorSubcoreMesh(core_axis_name="c", subcore_axis_name="s")
    cparams = pltpu.CompilerParams(use_tc_tiling_on_sc=False)

    t_flat = token_reservoir_lookup.reshape(-1)
    t_pad = jnp.concatenate(
        [t_flat,
         jnp.zeros((_VOCAB_PAD - _VOCAB) * _RESERVOIR, jnp.int32)]
    ).reshape(_N_CHUNKS_PAD, _CHUNK_ROWS, _RESERVOIR).transpose(0, 2, 1)

    build = functools.partial(
        pl.kernel,
        mesh=mesh,
        compiler_params=cparams,
        out_type=jax.ShapeDtypeStruct((_VOCAB_PAD, _FEATURES), jnp.float32),
        scratch_types=[
            pltpu.VMEM((_P1_NBUF, _RESERVOIR, _CHUNK_ROWS), jnp.int32),
            pltpu.VMEM((_P1_NBUF, _CHUNK_ROWS, _FEATURES), jnp.float32),
            [pltpu.SemaphoreType.DMA] * _P1_NBUF,
            [pltpu.SemaphoreType.DMA] * _P1_NBUF,
            [pltpu.SemaphoreType.DMA] * _P1_NBUF,
        ],
    )(_build_reduced_table)
    reduced = build(t_pad, embedding_table)

    apply_k = functools.partial(
        pl.kernel,
        mesh=mesh,
        compiler_params=cparams,
        out_type=jax.ShapeDtypeStruct((_N_TOKENS, _FEATURES), jnp.float32),
        scratch_types=[
            pltpu.VMEM((_P2_NBUF, _TOK_CHUNK), jnp.int32),
            pltpu.VMEM((_P2_NBUF, _TOK_CHUNK, _FEATURES), jnp.float32),
            [pltpu.SemaphoreType.DMA] * _P2_NBUF,
            [pltpu.SemaphoreType.DMA] * _P2_NBUF,
            [pltpu.SemaphoreType.DMA] * _P2_NBUF,
        ],
    )(_apply_table)
    out = apply_k(base_indices.reshape(-1), reduced)
    return out.reshape(*base_indices.shape, _FEATURES)


# speedup vs baseline: 1.8201x; 1.0113x over previous
import functools

import jax
import jax.numpy as jnp
from jax import lax
from jax.experimental import pallas as pl
from jax.experimental.pallas import tpu as pltpu
from jax.experimental.pallas import tpu_sc as plsc

NC = 2
NS = 16
NW = NC * NS
L = 16

_VOCAB = 100000
_RESERVOIR = 8
_FEATURES = 64
_N_TOKENS = 4096 * 50
_NDREG = _FEATURES // L

_CHUNK_ROWS = 64
_CHUNK_IDX = _CHUNK_ROWS * _RESERVOIR
_P1_CHUNKS_PER_W = 49
_N_CHUNKS_PAD = _P1_CHUNKS_PER_W * NW
_VOCAB_PAD = _N_CHUNKS_PAD * _CHUNK_ROWS
_P1_NBUF = 7
_P1_LOOK = 3
_P1_OUTER = _P1_CHUNKS_PER_W // _P1_NBUF

_TOK_CHUNK = 128
_TOK_PER_W = _N_TOKENS // NW
_P2_CHUNKS_PER_W = _TOK_PER_W // _TOK_CHUNK
_P2_NBUF = 10
_P2_LOOK = 4
_P2_OUTER = _P2_CHUNKS_PER_W // _P2_NBUF


def _build_reduced_table(t3_hbm, e_hbm, r_hbm, idx_v, acc_v, gsem, wsem,
                         isem):
    wid = lax.axis_index("s") * NC + lax.axis_index("c")
    c0 = wid * _P1_CHUNKS_PER_W

    def idx_cp(chunk, b):
        return pltpu.make_async_copy(t3_hbm.at[chunk], idx_v.at[b], isem[b])

    def fire_gathers(b):
        zero = jnp.zeros((L,), jnp.float32)
        for j in range(_CHUNK_ROWS):
            for d in range(_NDREG):
                acc_v[b, j, pl.ds(d * L, L)] = zero
        for r in range(_RESERVOIR):
            pltpu.async_copy(e_hbm.at[idx_v.at[b, r]], acc_v.at[b], gsem[b],
                             add=True)

    def wait_gathers(b):
        for _r in range(_RESERVOIR):
            pltpu.make_async_copy(e_hbm.at[idx_v.at[b, 0]], acc_v.at[b],
                                  gsem[b]).wait()

    def wback(chunk, b):
        return pltpu.make_async_copy(
            acc_v.at[b], r_hbm.at[pl.ds(chunk * _CHUNK_ROWS, _CHUNK_ROWS)],
            wsem[b])

    for b in range(_P1_NBUF):
        idx_cp(c0 + b, b).start()
    for b in range(_P1_LOOK):
        idx_cp(c0 + b, b).wait()
        fire_gathers(b)

    def body(o, _):
        for b in range(_P1_NBUF):
            i = o * _P1_NBUF + b
            c = c0 + i
            wait_gathers(b)
            @pl.when(i + _P1_NBUF < _P1_CHUNKS_PER_W)
            def _():
                idx_cp(c + _P1_NBUF, b).start()
            bj = (b + _P1_LOOK) % _P1_NBUF
            @pl.when(i + _P1_LOOK < _P1_CHUNKS_PER_W)
            def _():
                idx_cp(c + _P1_LOOK, bj).wait()
                @pl.when(i + _P1_LOOK >= _P1_NBUF)
                def _():
                    wback(c + _P1_LOOK - _P1_NBUF, bj).wait()
                fire_gathers(bj)
            wback(c, b).start()
        return 0

    lax.fori_loop(0, _P1_OUTER, body, 0)
    for b in range(_P1_NBUF):
        wback(c0 + _P1_CHUNKS_PER_W - _P1_NBUF + b, b).wait()


def _apply_table(base_hbm, r_hbm, out_hbm, idx_v, rows_v, gsem, wsem, isem):
    wid = lax.axis_index("s") * NC + lax.axis_index("c")
    t0w = wid * _TOK_PER_W

    def idx_cp(i, b):
        return pltpu.make_async_copy(
            base_hbm.at[pl.ds(t0w + i * _TOK_CHUNK, _TOK_CHUNK)],
            idx_v.at[b], isem[b])

    def gather(b):
        return pltpu.make_async_copy(r_hbm.at[idx_v.at[b]], rows_v.at[b],
                                     gsem[b])

    def wback(i, b):
        return pltpu.make_async_copy(
            rows_v.at[b],
            out_hbm.at[pl.ds(t0w + i * _TOK_CHUNK, _TOK_CHUNK)], wsem[b])

    for b in range(_P2_NBUF):
        idx_cp(b, b).start()
    for b in range(_P2_LOOK):
        idx_cp(b, b).wait()
        gather(b).start()

    def body(o, _):
        for b in range(_P2_NBUF):
            i = o * _P2_NBUF + b
            gather(b).wait()
            @pl.when(i + _P2_NBUF < _P2_CHUNKS_PER_W)
            def _():
                idx_cp(i + _P2_NBUF, b).start()
            bj = (b + _P2_LOOK) % _P2_NBUF
            @pl.when(i + _P2_LOOK < _P2_CHUNKS_PER_W)
            def _():
                idx_cp(i + _P2_LOOK, bj).wait()
                @pl.when(i + _P2_LOOK >= _P2_NBUF)
                def _():
                    wback(i + _P2_LOOK - _P2_NBUF, bj).wait()
                gather(bj).start()
            wback(i, b).start()
        return 0

    lax.fori_loop(0, _P2_OUTER, body, 0)
    for b in range(_P2_NBUF):
        wback(_P2_CHUNKS_PER_W - _P2_NBUF + b, b).wait()


@jax.jit
def kernel(base_indices, token_reservoir_lookup, embedding_table):
    mesh = plsc.VectorSubcoreMesh(core_axis_name="c", subcore_axis_name="s")
    cparams = pltpu.CompilerParams(use_tc_tiling_on_sc=False)

    t_flat = token_reservoir_lookup.reshape(-1)
    t_pad = jnp.concatenate(
        [t_flat,
         jnp.zeros((_VOCAB_PAD - _VOCAB) * _RESERVOIR, jnp.int32)]
    ).reshape(_N_CHUNKS_PAD, _CHUNK_ROWS, _RESERVOIR).transpose(0, 2, 1)

    build = functools.partial(
        pl.kernel,
        mesh=mesh,
        compiler_params=cparams,
        out_type=jax.ShapeDtypeStruct((_VOCAB_PAD, _FEATURES), jnp.float32),
        scratch_types=[
            pltpu.VMEM((_P1_NBUF, _RESERVOIR, _CHUNK_ROWS), jnp.int32),
            pltpu.VMEM((_P1_NBUF, _CHUNK_ROWS, _FEATURES), jnp.float32),
            [pltpu.SemaphoreType.DMA] * _P1_NBUF,
            [pltpu.SemaphoreType.DMA] * _P1_NBUF,
            [pltpu.SemaphoreType.DMA] * _P1_NBUF,
        ],
    )(_build_reduced_table)
    reduced = build(t_pad, embedding_table)

    apply_k = functools.partial(
        pl.kernel,
        mesh=mesh,
        compiler_params=cparams,
        out_type=jax.ShapeDtypeStruct((_N_TOKENS, _FEATURES), jnp.float32),
        scratch_types=[
            pltpu.VMEM((_P2_NBUF, _TOK_CHUNK), jnp.int32),
            pltpu.VMEM((_P2_NBUF, _TOK_CHUNK, _FEATURES), jnp.float32),
            [pltpu.SemaphoreType.DMA] * _P2_NBUF,
            [pltpu.SemaphoreType.DMA] * _P2_NBUF,
            [pltpu.SemaphoreType.DMA] * _P2_NBUF,
        ],
    )(_apply_table)
    out = apply_k(base_indices.reshape(-1), reduced)
    return out.reshape(*base_indices.shape, _FEATURES)


# speedup vs baseline: 2.0821x; 1.1440x over previous
import functools

import jax
import jax.numpy as jnp
from jax import lax
from jax.experimental import pallas as pl
from jax.experimental.pallas import tpu as pltpu
from jax.experimental.pallas import tpu_sc as plsc

NC = 2
NS = 16
NW = NC * NS
L = 16

_VOCAB = 100000
_RESERVOIR = 8
_FEATURES = 64
_N_TOKENS = 4096 * 50
_NDREG = _FEATURES // L

_CHUNK_ROWS = 64
_CHUNK_IDX = _CHUNK_ROWS * _RESERVOIR
_P1_CHUNKS_PER_W = 49
_N_CHUNKS_PAD = _P1_CHUNKS_PER_W * NW
_VOCAB_PAD = _N_CHUNKS_PAD * _CHUNK_ROWS
_P1_NBUF = 7
_P1_LOOK = 3
_P1_OUTER = _P1_CHUNKS_PER_W // _P1_NBUF

_TOK_CHUNK = 128
_TOK_PER_W = _N_TOKENS // NW
_P2_CHUNKS_PER_W = _TOK_PER_W // _TOK_CHUNK
_P2_NBUF = 10
_P2_LOOK = 4
_P2_OUTER = _P2_CHUNKS_PER_W // _P2_NBUF


def _build_reduced_table(t2d_hbm, e_hbm, r_hbm, idx_v, idxT_v, acc_v, gsem,
                         wsem, isem):
    wid = lax.axis_index("s") * NC + lax.axis_index("c")
    c0 = wid * _P1_CHUNKS_PER_W

    def idx_cp(chunk, b):
        return pltpu.make_async_copy(t2d_hbm.at[chunk], idx_v.at[b], isem[b])

    def transpose_idx(b):
        iota8 = jax.lax.broadcasted_iota(jnp.int32, (L,), 0) * _RESERVOIR
        for r in range(_RESERVOIR):
            for q in range(_CHUNK_ROWS // L):
                vec = iota8 + (q * L * _RESERVOIR + r)
                idxT_v[b, r, pl.ds(q * L, L)] = plsc.load_gather(
                    idx_v.at[b], [vec])

    def fire_gathers(b):
        zero = jnp.zeros((L,), jnp.float32)
        for j in range(_CHUNK_ROWS):
            for d in range(_NDREG):
                acc_v[b, j, pl.ds(d * L, L)] = zero
        for r in range(_RESERVOIR):
            pltpu.async_copy(e_hbm.at[idxT_v.at[b, r]], acc_v.at[b], gsem[b],
                             add=True)

    def wait_gathers(b):
        for _r in range(_RESERVOIR):
            pltpu.make_async_copy(e_hbm.at[idxT_v.at[b, 0]], acc_v.at[b],
                                  gsem[b]).wait()

    def wback(chunk, b):
        return pltpu.make_async_copy(
            acc_v.at[b], r_hbm.at[pl.ds(chunk * _CHUNK_ROWS, _CHUNK_ROWS)],
            wsem[b])

    for b in range(_P1_NBUF):
        idx_cp(c0 + b, b).start()
    for b in range(_P1_LOOK):
        idx_cp(c0 + b, b).wait()
        transpose_idx(b)
        fire_gathers(b)

    def body(o, _):
        for b in range(_P1_NBUF):
            i = o * _P1_NBUF + b
            c = c0 + i
            wait_gathers(b)
            @pl.when(i + _P1_NBUF < _P1_CHUNKS_PER_W)
            def _():
                idx_cp(c + _P1_NBUF, b).start()
            bj = (b + _P1_LOOK) % _P1_NBUF
            @pl.when(i + _P1_LOOK < _P1_CHUNKS_PER_W)
            def _():
                idx_cp(c + _P1_LOOK, bj).wait()
                transpose_idx(bj)
                @pl.when(i + _P1_LOOK >= _P1_NBUF)
                def _():
                    wback(c + _P1_LOOK - _P1_NBUF, bj).wait()
                fire_gathers(bj)
            wback(c, b).start()
        return 0

    lax.fori_loop(0, _P1_OUTER, body, 0)
    for b in range(_P1_NBUF):
        wback(c0 + _P1_CHUNKS_PER_W - _P1_NBUF + b, b).wait()


def _apply_table(base_hbm, r_hbm, out_hbm, idx_v, rows_v, gsem, wsem, isem):
    wid = lax.axis_index("s") * NC + lax.axis_index("c")
    t0w = wid * _TOK_PER_W

    def idx_cp(i, b):
        return pltpu.make_async_copy(
            base_hbm.at[pl.ds(t0w + i * _TOK_CHUNK, _TOK_CHUNK)],
            idx_v.at[b], isem[b])

    def gather(b):
        return pltpu.make_async_copy(r_hbm.at[idx_v.at[b]], rows_v.at[b],
                                     gsem[b])

    def wback(i, b):
        return pltpu.make_async_copy(
            rows_v.at[b],
            out_hbm.at[pl.ds(t0w + i * _TOK_CHUNK, _TOK_CHUNK)], wsem[b])

    for b in range(_P2_NBUF):
        idx_cp(b, b).start()
    for b in range(_P2_LOOK):
        idx_cp(b, b).wait()
        gather(b).start()

    def body(o, _):
        for b in range(_P2_NBUF):
            i = o * _P2_NBUF + b
            gather(b).wait()
            @pl.when(i + _P2_NBUF < _P2_CHUNKS_PER_W)
            def _():
                idx_cp(i + _P2_NBUF, b).start()
            bj = (b + _P2_LOOK) % _P2_NBUF
            @pl.when(i + _P2_LOOK < _P2_CHUNKS_PER_W)
            def _():
                idx_cp(i + _P2_LOOK, bj).wait()
                @pl.when(i + _P2_LOOK >= _P2_NBUF)
                def _():
                    wback(i + _P2_LOOK - _P2_NBUF, bj).wait()
                gather(bj).start()
            wback(i, b).start()
        return 0

    lax.fori_loop(0, _P2_OUTER, body, 0)
    for b in range(_P2_NBUF):
        wback(_P2_CHUNKS_PER_W - _P2_NBUF + b, b).wait()


@jax.jit
def kernel(base_indices, token_reservoir_lookup, embedding_table):
    mesh = plsc.VectorSubcoreMesh(core_axis_name="c", subcore_axis_name="s")
    cparams = pltpu.CompilerParams(use_tc_tiling_on_sc=False,
                                   needs_layout_passes=False)

    t_flat = token_reservoir_lookup.reshape(-1)
    t_pad = jnp.concatenate(
        [t_flat,
         jnp.zeros((_VOCAB_PAD - _VOCAB) * _RESERVOIR, jnp.int32)]
    ).reshape(_N_CHUNKS_PAD, _CHUNK_IDX)

    build = functools.partial(
        pl.kernel,
        mesh=mesh,
        compiler_params=cparams,
        out_type=jax.ShapeDtypeStruct((_VOCAB_PAD, _FEATURES), jnp.float32),
        scratch_types=[
            pltpu.VMEM((_P1_NBUF, _CHUNK_IDX), jnp.int32),
            pltpu.VMEM((_P1_NBUF, _RESERVOIR, _CHUNK_ROWS), jnp.int32),
            pltpu.VMEM((_P1_NBUF, _CHUNK_ROWS, _FEATURES), jnp.float32),
            [pltpu.SemaphoreType.DMA] * _P1_NBUF,
            [pltpu.SemaphoreType.DMA] * _P1_NBUF,
            [pltpu.SemaphoreType.DMA] * _P1_NBUF,
        ],
    )(_build_reduced_table)
    reduced = build(t_pad, embedding_table)

    apply_k = functools.partial(
        pl.kernel,
        mesh=mesh,
        compiler_params=cparams,
        out_type=jax.ShapeDtypeStruct((_N_TOKENS, _FEATURES), jnp.float32),
        scratch_types=[
            pltpu.VMEM((_P2_NBUF, _TOK_CHUNK), jnp.int32),
            pltpu.VMEM((_P2_NBUF, _TOK_CHUNK, _FEATURES), jnp.float32),
            [pltpu.SemaphoreType.DMA] * _P2_NBUF,
            [pltpu.SemaphoreType.DMA] * _P2_NBUF,
            [pltpu.SemaphoreType.DMA] * _P2_NBUF,
        ],
    )(_apply_table)
    out = apply_k(base_indices.reshape(-1), reduced)
    return out.reshape(*base_indices.shape, _FEATURES)


# speedup vs baseline: 2.4320x; 1.1680x over previous
import functools

import jax
import jax.numpy as jnp
from jax import lax
from jax.experimental import pallas as pl
from jax.experimental.pallas import tpu as pltpu
from jax.experimental.pallas import tpu_sc as plsc

NC = 2
NS = 16
NW = NC * NS
L = 16

_VOCAB = 100000
_RESERVOIR = 8
_FEATURES = 64
_BATCH = 4096
_HIST = 50

_CHUNK_ROWS = 64
_P1_CHUNKS_PER_W = 49
_P1_ROWS_PER_W = _P1_CHUNKS_PER_W * _CHUNK_ROWS
_P1_NBUF = 7
_P1_LOOK = 3
_P1_OUTER = _P1_CHUNKS_PER_W // _P1_NBUF

_ROWS_PER_CHUNK = 8
_P2_ROWS_PER_W = _BATCH // NW
_P2_CHUNKS_PER_W = _P2_ROWS_PER_W // _ROWS_PER_CHUNK
_P2_NBUF = 4
_P2_LOOK = 2
_P2_OUTER = _P2_CHUNKS_PER_W // _P2_NBUF


def _build_reduced_table(t_hbm, e_hbm, r_hbm, idx_v, idxT_v, acc_v, gsem,
                         wsem, isem):
    wid = lax.axis_index("s") * NC + lax.axis_index("c")
    row0 = wid * _P1_ROWS_PER_W

    def chunk_start(i):
        return jnp.minimum(row0 + i * _CHUNK_ROWS, _VOCAB - _CHUNK_ROWS)

    def idx_cp(i, b):
        return pltpu.make_async_copy(
            t_hbm.at[pl.ds(chunk_start(i), _CHUNK_ROWS), :], idx_v.at[b],
            isem[b])

    def transpose_idx(b):
        iota = jax.lax.broadcasted_iota(jnp.int32, (L,), 0)
        for r in range(_RESERVOIR):
            col = jnp.full((L,), r, jnp.int32)
            for q in range(_CHUNK_ROWS // L):
                idxT_v[b, r, pl.ds(q * L, L)] = plsc.load_gather(
                    idx_v.at[b], [iota + q * L, col])

    def fire_gathers(b):
        zero = jnp.zeros((L,), jnp.float32)
        for j in range(_CHUNK_ROWS):
            for d in range(_FEATURES // L):
                acc_v[b, j, pl.ds(d * L, L)] = zero
        for r in range(_RESERVOIR):
            pltpu.async_copy(e_hbm.at[idxT_v.at[b, r]], acc_v.at[b], gsem[b],
                             add=True)

    def wait_gathers(b):
        for _r in range(_RESERVOIR):
            pltpu.make_async_copy(e_hbm.at[idxT_v.at[b, 0]], acc_v.at[b],
                                  gsem[b]).wait()

    def wback(i, b):
        return pltpu.make_async_copy(
            acc_v.at[b], r_hbm.at[pl.ds(chunk_start(i), _CHUNK_ROWS)],
            wsem[b])

    for b in range(_P1_NBUF):
        idx_cp(b, b).start()
    for b in range(_P1_LOOK):
        idx_cp(b, b).wait()
        transpose_idx(b)
        fire_gathers(b)

    def body(o, _):
        for b in range(_P1_NBUF):
            i = o * _P1_NBUF + b
            wait_gathers(b)
            @pl.when(i + _P1_NBUF < _P1_CHUNKS_PER_W)
            def _():
                idx_cp(i + _P1_NBUF, b).start()
            bj = (b + _P1_LOOK) % _P1_NBUF
            @pl.when(i + _P1_LOOK < _P1_CHUNKS_PER_W)
            def _():
                idx_cp(i + _P1_LOOK, bj).wait()
                transpose_idx(bj)
                @pl.when(i + _P1_LOOK >= _P1_NBUF)
                def _():
                    wback(i + _P1_LOOK - _P1_NBUF, bj).wait()
                fire_gathers(bj)
            wback(i, b).start()
        return 0

    lax.fori_loop(0, _P1_OUTER, body, 0)
    for b in range(_P1_NBUF):
        wback(_P1_CHUNKS_PER_W - _P1_NBUF + b, b).wait()


def _apply_table(base_hbm, r_hbm, out_hbm, idx_v, rows_v, gsem, wsem, isem):
    wid = lax.axis_index("s") * NC + lax.axis_index("c")
    row0 = wid * _P2_ROWS_PER_W

    def idx_cp(i, b):
        return pltpu.make_async_copy(
            base_hbm.at[pl.ds(row0 + i * _ROWS_PER_CHUNK, _ROWS_PER_CHUNK), :],
            idx_v.at[b], isem[b])

    def fire_gathers(b):
        for ci in range(_ROWS_PER_CHUNK):
            pltpu.async_copy(r_hbm.at[idx_v.at[b, ci]], rows_v.at[b, ci],
                             gsem[b])

    def wait_gathers(b):
        for _ci in range(_ROWS_PER_CHUNK):
            pltpu.make_async_copy(r_hbm.at[idx_v.at[b, 0]], rows_v.at[b, 0],
                                  gsem[b]).wait()

    def wback(i, b):
        return pltpu.make_async_copy(
            rows_v.at[b],
            out_hbm.at[pl.ds(row0 + i * _ROWS_PER_CHUNK, _ROWS_PER_CHUNK)],
            wsem[b])

    for b in range(_P2_NBUF):
        idx_cp(b, b).start()
    for b in range(_P2_LOOK):
        idx_cp(b, b).wait()
        fire_gathers(b)

    def body(o, _):
        for b in range(_P2_NBUF):
            i = o * _P2_NBUF + b
            wait_gathers(b)
            @pl.when(i + _P2_NBUF < _P2_CHUNKS_PER_W)
            def _():
                idx_cp(i + _P2_NBUF, b).start()
            bj = (b + _P2_LOOK) % _P2_NBUF
            @pl.when(i + _P2_LOOK < _P2_CHUNKS_PER_W)
            def _():
                idx_cp(i + _P2_LOOK, bj).wait()
                @pl.when(i + _P2_LOOK >= _P2_NBUF)
                def _():
                    wback(i + _P2_LOOK - _P2_NBUF, bj).wait()
                fire_gathers(bj)
            wback(i, b).start()
        return 0

    lax.fori_loop(0, _P2_OUTER, body, 0)
    for b in range(_P2_NBUF):
        wback(_P2_CHUNKS_PER_W - _P2_NBUF + b, b).wait()


@jax.jit
def kernel(base_indices, token_reservoir_lookup, embedding_table):
    mesh = plsc.VectorSubcoreMesh(core_axis_name="c", subcore_axis_name="s")
    cparams = pltpu.CompilerParams(use_tc_tiling_on_sc=False,
                                   needs_layout_passes=False)

    build = functools.partial(
        pl.kernel,
        mesh=mesh,
        compiler_params=cparams,
        out_type=jax.ShapeDtypeStruct((_VOCAB, _FEATURES), jnp.float32),
        scratch_types=[
            pltpu.VMEM((_P1_NBUF, _CHUNK_ROWS, _RESERVOIR), jnp.int32),
            pltpu.VMEM((_P1_NBUF, _RESERVOIR, _CHUNK_ROWS), jnp.int32),
            pltpu.VMEM((_P1_NBUF, _CHUNK_ROWS, _FEATURES), jnp.float32),
            [pltpu.SemaphoreType.DMA] * _P1_NBUF,
            [pltpu.SemaphoreType.DMA] * _P1_NBUF,
            [pltpu.SemaphoreType.DMA] * _P1_NBUF,
        ],
    )(_build_reduced_table)
    reduced = build(token_reservoir_lookup, embedding_table)

    apply_k = functools.partial(
        pl.kernel,
        mesh=mesh,
        compiler_params=cparams,
        out_type=jax.ShapeDtypeStruct((_BATCH, _HIST, _FEATURES),
                                      jnp.float32),
        scratch_types=[
            pltpu.VMEM((_P2_NBUF, _ROWS_PER_CHUNK, _HIST), jnp.int32),
            pltpu.VMEM((_P2_NBUF, _ROWS_PER_CHUNK, _HIST, _FEATURES),
                       jnp.float32),
            [pltpu.SemaphoreType.DMA] * _P2_NBUF,
            [pltpu.SemaphoreType.DMA] * _P2_NBUF,
            [pltpu.SemaphoreType.DMA] * _P2_NBUF,
        ],
    )(_apply_table)
    return apply_k(base_indices, reduced)


# speedup vs baseline: 2.7919x; 1.1480x over previous
import functools

import jax
import jax.numpy as jnp
from jax import lax
from jax.experimental import pallas as pl
from jax.experimental.pallas import tpu as pltpu
from jax.experimental.pallas import tpu_sc as plsc

NC = 2
NS = 16
NW = NC * NS
L = 16

_VOCAB = 100000
_RESERVOIR = 8
_FEATURES = 64
_BATCH = 4096
_HIST = 50

_CHUNK_ROWS = 64
_P1_CHUNKS_PER_W = 49
_P1_ROWS_PER_W = _P1_CHUNKS_PER_W * _CHUNK_ROWS
_P1_NBUF = 7
_P1_LOOK = 3
_P1_OUTER = _P1_CHUNKS_PER_W // _P1_NBUF

_ROWS_PER_CHUNK = 8
_P2_ROWS_PER_W = _BATCH // NW
_P2_CHUNKS_PER_W = _P2_ROWS_PER_W // _ROWS_PER_CHUNK
_P2_NBUF = 4
_P2_LOOK = 2
_P2_OUTER = _P2_CHUNKS_PER_W // _P2_NBUF


def _build_reduced_table(tt_hbm, e_hbm, r_hbm, idxT_v, acc_v, gsem,
                         wsem, isem):
    wid = lax.axis_index("s") * NC + lax.axis_index("c")
    row0 = wid * _P1_ROWS_PER_W

    def chunk_start(i):
        return jnp.minimum(row0 + i * _CHUNK_ROWS, _VOCAB - _CHUNK_ROWS)

    def idx_start(i, b):
        s = chunk_start(i)
        for r in range(_RESERVOIR):
            pltpu.async_copy(tt_hbm.at[r, pl.ds(s, _CHUNK_ROWS)],
                             idxT_v.at[b, r], isem[b])

    def idx_wait(b):
        for _r in range(_RESERVOIR):
            pltpu.make_async_copy(tt_hbm.at[0, pl.ds(0, _CHUNK_ROWS)],
                                  idxT_v.at[b, 0], isem[b]).wait()

    def fire_gathers(b):
        zero = jnp.zeros((L,), jnp.float32)
        for j in range(_CHUNK_ROWS):
            for d in range(_FEATURES // L):
                acc_v[b, j, pl.ds(d * L, L)] = zero
        for r in range(_RESERVOIR):
            pltpu.async_copy(e_hbm.at[idxT_v.at[b, r]], acc_v.at[b], gsem[b],
                             add=True)

    def wait_gathers(b):
        for _r in range(_RESERVOIR):
            pltpu.make_async_copy(e_hbm.at[idxT_v.at[b, 0]], acc_v.at[b],
                                  gsem[b]).wait()

    def wback(i, b):
        return pltpu.make_async_copy(
            acc_v.at[b], r_hbm.at[pl.ds(chunk_start(i), _CHUNK_ROWS)],
            wsem[b])

    for b in range(_P1_NBUF):
        idx_start(b, b)
    for b in range(_P1_LOOK):
        idx_wait(b)
        fire_gathers(b)

    def body(o, _):
        for b in range(_P1_NBUF):
            i = o * _P1_NBUF + b
            wait_gathers(b)
            @pl.when(i + _P1_NBUF < _P1_CHUNKS_PER_W)
            def _():
                idx_start(i + _P1_NBUF, b)
            bj = (b + _P1_LOOK) % _P1_NBUF
            @pl.when(i + _P1_LOOK < _P1_CHUNKS_PER_W)
            def _():
                idx_wait(bj)
                @pl.when(i + _P1_LOOK >= _P1_NBUF)
                def _():
                    wback(i + _P1_LOOK - _P1_NBUF, bj).wait()
                fire_gathers(bj)
            wback(i, b).start()
        return 0

    lax.fori_loop(0, _P1_OUTER, body, 0)
    for b in range(_P1_NBUF):
        wback(_P1_CHUNKS_PER_W - _P1_NBUF + b, b).wait()


def _apply_table(base_hbm, r_hbm, out_hbm, idx_v, rows_v, gsem, wsem, isem):
    wid = lax.axis_index("s") * NC + lax.axis_index("c")
    row0 = wid * _P2_ROWS_PER_W

    def idx_cp(i, b):
        return pltpu.make_async_copy(
            base_hbm.at[pl.ds(row0 + i * _ROWS_PER_CHUNK, _ROWS_PER_CHUNK), :],
            idx_v.at[b], isem[b])

    def fire_gathers(b):
        for ci in range(_ROWS_PER_CHUNK):
            pltpu.async_copy(r_hbm.at[idx_v.at[b, ci]], rows_v.at[b, ci],
                             gsem[b])

    def wait_gathers(b):
        for _ci in range(_ROWS_PER_CHUNK):
            pltpu.make_async_copy(r_hbm.at[idx_v.at[b, 0]], rows_v.at[b, 0],
                                  gsem[b]).wait()

    def wback(i, b):
        return pltpu.make_async_copy(
            rows_v.at[b],
            out_hbm.at[pl.ds(row0 + i * _ROWS_PER_CHUNK, _ROWS_PER_CHUNK)],
            wsem[b])

    for b in range(_P2_NBUF):
        idx_cp(b, b).start()
    for b in range(_P2_LOOK):
        idx_cp(b, b).wait()
        fire_gathers(b)

    def body(o, _):
        for b in range(_P2_NBUF):
            i = o * _P2_NBUF + b
            wait_gathers(b)
            @pl.when(i + _P2_NBUF < _P2_CHUNKS_PER_W)
            def _():
                idx_cp(i + _P2_NBUF, b).start()
            bj = (b + _P2_LOOK) % _P2_NBUF
            @pl.when(i + _P2_LOOK < _P2_CHUNKS_PER_W)
            def _():
                idx_cp(i + _P2_LOOK, bj).wait()
                @pl.when(i + _P2_LOOK >= _P2_NBUF)
                def _():
                    wback(i + _P2_LOOK - _P2_NBUF, bj).wait()
                fire_gathers(bj)
            wback(i, b).start()
        return 0

    lax.fori_loop(0, _P2_OUTER, body, 0)
    for b in range(_P2_NBUF):
        wback(_P2_CHUNKS_PER_W - _P2_NBUF + b, b).wait()


@jax.jit
def kernel(base_indices, token_reservoir_lookup, embedding_table):
    mesh = plsc.VectorSubcoreMesh(core_axis_name="c", subcore_axis_name="s")
    cparams = pltpu.CompilerParams(use_tc_tiling_on_sc=False,
                                   needs_layout_passes=False)

    build = functools.partial(
        pl.kernel,
        mesh=mesh,
        compiler_params=cparams,
        out_type=jax.ShapeDtypeStruct((_VOCAB, _FEATURES), jnp.float32),
        scratch_types=[
            pltpu.VMEM((_P1_NBUF, _RESERVOIR, _CHUNK_ROWS), jnp.int32),
            pltpu.VMEM((_P1_NBUF, _CHUNK_ROWS, _FEATURES), jnp.float32),
            [pltpu.SemaphoreType.DMA] * _P1_NBUF,
            [pltpu.SemaphoreType.DMA] * _P1_NBUF,
            [pltpu.SemaphoreType.DMA] * _P1_NBUF,
        ],
    )(_build_reduced_table)
    reduced = build(token_reservoir_lookup.T, embedding_table)

    apply_k = functools.partial(
        pl.kernel,
        mesh=mesh,
        compiler_params=cparams,
        out_type=jax.ShapeDtypeStruct((_BATCH, _HIST, _FEATURES),
                                      jnp.float32),
        scratch_types=[
            pltpu.VMEM((_P2_NBUF, _ROWS_PER_CHUNK, _HIST), jnp.int32),
            pltpu.VMEM((_P2_NBUF, _ROWS_PER_CHUNK, _HIST, _FEATURES),
                       jnp.float32),
            [pltpu.SemaphoreType.DMA] * _P2_NBUF,
            [pltpu.SemaphoreType.DMA] * _P2_NBUF,
            [pltpu.SemaphoreType.DMA] * _P2_NBUF,
        ],
    )(_apply_table)
    return apply_k(base_indices, reduced)
